# baseline (speedup 1.0000x reference)
import functools

import jax
import jax.numpy as jnp
from jax import lax
from jax.experimental import pallas as pl
from jax.experimental.pallas import tpu as pltpu
from jax.experimental.pallas import tpu_sc as plsc

S, D, H, E = 2048, 1024, 2048, 8
PREC = jax.lax.Precision.DEFAULT
BT = 256
NT = S // BT
BLK = 512
NBLK = 16
TOT = NBLK * BLK
NP = 2 * S


def _router_body(xpad_ref, w_ref, b1_ref, v_ref, b2_ref,
                 idx_ref, gate_ref, cnt_ref):
    t = pl.program_id(0)
    base = t * BT
    win = xpad_ref[pl.ds(base, BT + 16), :]
    g = None
    for k in range(3):
        xs = win[6 + k:6 + k + BT + 8, :]
        tt = jax.lax.dot_general(xs, w_ref[k], (((1,), (0,)), ((), ())),
                                 precision=PREC,
                                 preferred_element_type=jnp.float32)
        g = tt if g is None else g + tt
    g = g + b1_ref[...]
    g = 0.5 * g * (1.0 + jax.lax.erf(g * 0.7071067811865476))
    rid = jax.lax.broadcasted_iota(jnp.int32, (BT + 8, 1), 0) + base - 1
    g = jnp.where((rid >= 0) & (rid < S), g, 0.0)
    lo = None
    for k in range(3):
        tt = jax.lax.dot_general(g[k:k + BT, :], v_ref[k],
                                 (((1,), (0,)), ((), ())), precision=PREC,
                                 preferred_element_type=jnp.float32)
        lo = tt if lo is None else lo + tt
    lo = lo + b2_ref[...]
    eid = jax.lax.broadcasted_iota(jnp.int32, (BT, E), 1)
    m1 = jnp.max(lo, axis=1, keepdims=True)
    a1 = jnp.min(jnp.where(lo >= m1, eid, E), axis=1, keepdims=True)
    lo2 = jnp.where(eid == a1, -jnp.inf, lo)
    m2 = jnp.max(lo2, axis=1, keepdims=True)
    a2 = jnp.min(jnp.where(lo2 >= m2, eid, E), axis=1, keepdims=True)
    g1 = 1.0 / (1.0 + jnp.exp(m2 - m1))
    idx_ref[...] = jnp.concatenate([a1, a2], axis=1)
    gate_ref[...] = jnp.concatenate([g1, 1.0 - g1], axis=1)
    c = jnp.sum((eid == a1).astype(jnp.int32) + (eid == a2).astype(jnp.int32),
                axis=0, keepdims=True)

    @pl.when(t == 0)
    def _():
        cnt_ref[...] = c

    @pl.when(t != 0)
    def _():
        cnt_ref[...] = cnt_ref[...] + c


def _scb_body(exp_hbm, gate_hbm, off_hbm, nb_hbm, nblk_hbm,
              tok_out, gate_out, p0_out, p1_out,
              expv, gatev, tokl, gatel, posl0, posl1,
              offv, nbv, nblkv, z512i, z512f):
    c = lax.axis_index("c")
    s = lax.axis_index("s")
    lanes = lax.iota(jnp.int32, 16)

    @pl.when((c == 0) & (s < E))
    def _():
        e = s
        pltpu.sync_copy(exp_hbm, expv)
        pltpu.sync_copy(gate_hbm, gatev)
        pltpu.sync_copy(off_hbm, offv)
        pltpu.sync_copy(nb_hbm, nbv)
        offsc = pl.multiple_of(jnp.sum(jnp.where(lanes == e, offv[...], 0)),
                               BLK)
        nbsc = jnp.sum(jnp.where(lanes == e, nbv[...], 0))

        def init(j, _):
            tokl[pl.ds(j * 16, 16)] = jnp.zeros((16,), jnp.int32)
            gatel[pl.ds(j * 16, 16)] = jnp.zeros((16,), jnp.float32)
            return 0

        lax.fori_loop(0, (NP + 16) // 16, init, 0)

        def initp(j, _):
            posl0[pl.ds(j * 16, 16)] = jnp.zeros((16,), jnp.int32)
            posl1[pl.ds(j * 16, 16)] = jnp.zeros((16,), jnp.int32)
            return 0

        lax.fori_loop(0, S // 16, initp, 0)
        evm = (lanes % 2) == 0

        def emit(i, wptr):
            ch = expv[pl.ds(i * 16, 16)]
            m = ch == e
            tok = lax.shift_right_logical(lanes + i * 16, 1)
            gt = gatev[pl.ds(i * 16, 16)]
            pos = wptr + plsc.cumsum(m.astype(jnp.int32)) - 1
            plsc.store_scatter(tokl, [pos], tok, mask=m)
            plsc.store_scatter(gatel, [pos], gt, mask=m)
            gslot = offsc + pos
            plsc.store_scatter(posl0, [tok], gslot, mask=m & evm)
            plsc.store_scatter(posl1, [tok], gslot, mask=m & (~evm))
            return wptr + jnp.sum(m.astype(jnp.int32))

        lax.fori_loop(0, NP // 16, emit, 0)
        pltpu.sync_copy(posl0, p0_out.at[e])
        pltpu.sync_copy(posl1, p1_out.at[e])
        for j in range(4):
            @pl.when(j < nbsc)
            def _():
                pltpu.sync_copy(tokl.at[pl.ds(j * BLK, BLK)],
                                tok_out.at[pl.ds(offsc + j * BLK, BLK)])
                pltpu.sync_copy(gatel.at[pl.ds(j * BLK, BLK)],
                                gate_out.at[pl.ds(offsc + j * BLK, BLK)])

    @pl.when((c == 0) & (s == E))
    def _():
        pltpu.sync_copy(nblk_hbm, nblkv)
        nblksc = jnp.sum(jnp.where(lanes == 0, nblkv[...], 0))

        def initz(j, _):
            z512i[pl.ds(j * 16, 16)] = jnp.zeros((16,), jnp.int32)
            z512f[pl.ds(j * 16, 16)] = jnp.zeros((16,), jnp.float32)
            return 0

        lax.fori_loop(0, BLK // 16, initz, 0)
        for j in range(NBLK):
            @pl.when(j >= nblksc)
            def _():
                pltpu.sync_copy(z512i, tok_out.at[pl.ds(j * BLK, BLK)])
                pltpu.sync_copy(z512f, gate_out.at[pl.ds(j * BLK, BLK)])


def _sc_sort(exp_flat, gate_flat, off16, nb16, nblk16):
    mesh = plsc.VectorSubcoreMesh(core_axis_name="c", subcore_axis_name="s")
    f = pl.kernel(
        _scb_body,
        out_type=(jax.ShapeDtypeStruct((TOT,), jnp.int32),
                  jax.ShapeDtypeStruct((TOT,), jnp.float32),
                  jax.ShapeDtypeStruct((E, S), jnp.int32),
                  jax.ShapeDtypeStruct((E, S), jnp.int32)),
        mesh=mesh,
        compiler_params=pltpu.CompilerParams(needs_layout_passes=False),
        scratch_types=[
            pltpu.VMEM((NP,), jnp.int32),
            pltpu.VMEM((NP,), jnp.float32),
            pltpu.VMEM((S + 16,), jnp.int32),
            pltpu.VMEM((S + 16,), jnp.float32),
            pltpu.VMEM((S,), jnp.int32),
            pltpu.VMEM((S,), jnp.int32),
            pltpu.VMEM((16,), jnp.int32),
            pltpu.VMEM((16,), jnp.int32),
            pltpu.VMEM((16,), jnp.int32),
            pltpu.VMEM((BLK,), jnp.int32),
            pltpu.VMEM((BLK,), jnp.float32),
        ],
    )
    return f(exp_flat, gate_flat, off16, nb16, nblk16)


def _sca_body(x_hbm, tok_hbm, xs_hbm, idx_v, rows_v, sem):
    w = lax.axis_index("s") * 2 + lax.axis_index("c")
    base = w * (TOT // 32)
    pltpu.sync_copy(tok_hbm.at[pl.ds(base, TOT // 32)], idx_v)
    for j in range(TOT // 32 // 64):
        pltpu.async_copy(x_hbm.at[idx_v.at[pl.ds(j * 64, 64)]], rows_v,
                         sem).wait()
        pltpu.sync_copy(rows_v, xs_hbm.at[pl.ds(base + j * 64, 64)])


def _sc_gather(xf, tok_s):
    mesh = plsc.VectorSubcoreMesh(core_axis_name="c", subcore_axis_name="s")
    f = pl.kernel(
        _sca_body,
        out_type=jax.ShapeDtypeStruct((TOT, D), jnp.float32),
        mesh=mesh,
        scratch_types=[
            pltpu.VMEM((TOT // 32,), jnp.int32),
            pltpu.VMEM((64, D), jnp.float32),
            pltpu.SemaphoreType.DMA,
        ],
    )
    return f(xf, tok_s)


def _moe_group_body(be_ref, nblk_ref, xs_ref, w1_ref, b1_ref, w2_ref, b2_ref,
                    g_ref, os_ref):
    i = pl.program_id(0)

    @pl.when(i < nblk_ref[0])
    def _():
        x = xs_ref[...]
        h = jax.lax.dot_general(x, w1_ref[0], (((1,), (0,)), ((), ())),
                                precision=PREC,
                                preferred_element_type=jnp.float32)
        h = h + b1_ref[0]
        h = jnp.where(h > 0, h, jnp.exp(jnp.minimum(h, 0.0)) - 1.0)
        o = jax.lax.dot_general(h, w2_ref[0], (((1,), (0,)), ((), ())),
                                precision=PREC,
                                preferred_element_type=jnp.float32)
        os_ref[...] = (o + b2_ref[0]) * g_ref[0]

    @pl.when(i >= nblk_ref[0])
    def _():
        os_ref[...] = jnp.zeros_like(os_ref)


def _moe_group(blkexp, nblk_arr, xs, w1, b1, w2, b2, gsort3):
    grid_spec = pltpu.PrefetchScalarGridSpec(
        num_scalar_prefetch=2,
        grid=(NBLK,),
        in_specs=[
            pl.BlockSpec((BLK, D), lambda i, be, nb: (i, 0)),
            pl.BlockSpec((1, D, H), lambda i, be, nb: (be[i], 0, 0)),
            pl.BlockSpec((1, 1, H), lambda i, be, nb: (be[i], 0, 0)),
            pl.BlockSpec((1, H, D), lambda i, be, nb: (be[i], 0, 0)),
            pl.BlockSpec((1, 1, D), lambda i, be, nb: (be[i], 0, 0)),
            pl.BlockSpec((1, BLK, 1), lambda i, be, nb: (i, 0, 0)),
        ],
        out_specs=pl.BlockSpec((BLK, D), lambda i, be, nb: (i, 0)),
    )
    return pl.pallas_call(
        _moe_group_body,
        grid_spec=grid_spec,
        out_shape=jax.ShapeDtypeStruct((TOT, D), jnp.float32),
        compiler_params=pltpu.CompilerParams(
            dimension_semantics=("arbitrary",)),
    )(blkexp, nblk_arr, xs, w1, b1[:, None, :], w2, b2[:, None, :], gsort3)


def _scd_body(os_hbm, p0_hbm, p1_hbm, y0_hbm, y1_hbm,
              pv, idxv, rows_v, sem):
    w = lax.axis_index("s") * 2 + lax.axis_index("c")
    base = w * (S // 32)
    for (p_hbm, y_hbm) in ((p0_hbm, y0_hbm), (p1_hbm, y1_hbm)):
        for e in range(E):
            pltpu.sync_copy(p_hbm.at[e, pl.ds(base, 64)], pv.at[e])
        for u in range(4):
            acc = jnp.zeros((16,), jnp.int32)
            for e in range(E):
                acc = acc + pv[e, pl.ds(u * 16, 16)]
            idxv[pl.ds(u * 16, 16)] = acc
        pltpu.async_copy(os_hbm.at[idxv], rows_v, sem).wait()
        pltpu.sync_copy(rows_v, y_hbm.at[pl.ds(base, 64)])


def _sc_combine(os, p0all, p1all):
    mesh = plsc.VectorSubcoreMesh(core_axis_name="c", subcore_axis_name="s")
    f = pl.kernel(
        _scd_body,
        out_type=(jax.ShapeDtypeStruct((S, D), jnp.float32),
                  jax.ShapeDtypeStruct((S, D), jnp.float32)),
        mesh=mesh,
        scratch_types=[
            pltpu.VMEM((E, 64), jnp.int32),
            pltpu.VMEM((64,), jnp.int32),
            pltpu.VMEM((64, D), jnp.float32),
            pltpu.SemaphoreType.DMA,
        ],
    )
    return f(os, p0all, p1all)


def _shared1_body(xpad_ref, w_ref, b_ref, h_ref):
    t = pl.program_id(0)
    base = t * BT
    win = xpad_ref[pl.ds(base, BT + 16), :]
    g = None
    for k in range(3):
        xs = win[7 + k:7 + k + BT, :]
        tt = jax.lax.dot_general(xs, w_ref[k], (((1,), (0,)), ((), ())),
                                 precision=PREC,
                                 preferred_element_type=jnp.float32)
        g = tt if g is None else g + tt
    g = g + b_ref[...]
    h_ref[...] = g * jax.nn.sigmoid(g)


def _shared2_body(h_ref, w_ref, b_ref, x_ref, y0_ref, y1_ref,
                  lng_ref, lnb_ref, o_ref):
    t = pl.program_id(0)
    base = t * BT
    win = h_ref[pl.ds(base, BT + 8), :]
    sh = None
    for k in range(3):
        tt = jax.lax.dot_general(win[k:k + BT, :], w_ref[k],
                                 (((1,), (0,)), ((), ())), precision=PREC,
                                 preferred_element_type=jnp.float32)
        sh = tt if sh is None else sh + tt
    z = x_ref[...] + y0_ref[...] + y1_ref[...] + sh + b_ref[...]
    mu = jnp.mean(z, axis=1, keepdims=True)
    zc = z - mu
    var = jnp.mean(zc * zc, axis=1, keepdims=True)
    o_ref[...] = zc * jax.lax.rsqrt(var + 1e-5) * lng_ref[...] + lnb_ref[...]


def kernel(x, rc1_w, rc1_b, rc2_w, rc2_b, exp_w1, exp_b1, exp_w2, exp_b2,
           sh1_w, sh1_b, sh2_w, sh2_b, ln_g, ln_b):
    xf = x.reshape(S, D)
    xpad = jnp.pad(xf, ((8, 8), (0, 0)))
    rc1 = jnp.transpose(rc1_w, (2, 1, 0))
    rc2 = jnp.transpose(rc2_w, (2, 1, 0))
    s1 = jnp.transpose(sh1_w, (2, 1, 0))
    s2 = jnp.transpose(sh2_w, (2, 1, 0))

    whole = lambda *shape: pl.BlockSpec(shape, lambda *a: tuple(0 for _ in shape))

    idx2, gate2, counts = pl.pallas_call(
        _router_body,
        grid=(NT,),
        in_specs=[
            whole(S + 16, D),
            whole(3, D, D),
            whole(1, D),
            whole(3, D, E),
            whole(1, E),
        ],
        out_specs=[
            pl.BlockSpec((BT, 2), lambda t: (t, 0)),
            pl.BlockSpec((BT, 2), lambda t: (t, 0)),
            pl.BlockSpec((1, E), lambda t: (0, 0)),
        ],
        out_shape=[
            jax.ShapeDtypeStruct((S, 2), jnp.int32),
            jax.ShapeDtypeStruct((S, 2), jnp.float32),
            jax.ShapeDtypeStruct((1, E), jnp.int32),
        ],
    )(xpad, rc1, rc1_b[None, :], rc2, rc2_b[None, :])

    counts8 = counts[0]
    nb = (counts8 + (BLK - 1)) // BLK
    cumnb = jnp.cumsum(nb)
    nblk = cumnb[E - 1]
    off16 = jnp.pad((cumnb - nb) * BLK, (0, 8)).astype(jnp.int32)
    nb16 = jnp.pad(nb, (0, 8)).astype(jnp.int32)
    nblk16 = jnp.pad(nblk[None], (0, 15)).astype(jnp.int32)
    bid = jnp.arange(NBLK, dtype=jnp.int32)
    blkexp = jnp.minimum(
        jnp.sum((bid[:, None] >= cumnb[None, :]).astype(jnp.int32), axis=1),
        E - 1).astype(jnp.int32)

    tok_s, gate_s, p0all, p1all = _sc_sort(idx2.reshape(-1), gate2.reshape(-1),
                                           off16, nb16, nblk16)
    xs = _sc_gather(xf, tok_s)
    os = _moe_group(blkexp, nblk[None].astype(jnp.int32), xs,
                    exp_w1, exp_b1, exp_w2, exp_b2,
                    gate_s.reshape(NBLK, BLK, 1))
    y0, y1 = _sc_combine(os, p0all, p1all)

    h = pl.pallas_call(
        _shared1_body,
        grid=(NT,),
        in_specs=[whole(S + 16, D), whole(3, D, H), whole(1, H)],
        out_specs=pl.BlockSpec((BT, H), lambda t: (t, 0)),
        out_shape=jax.ShapeDtypeStruct((S, H), jnp.float32),
    )(xpad, s1, sh1_b[None, :])

    hpad = jnp.pad(h, ((1, 7), (0, 0)))

    out = pl.pallas_call(
        _shared2_body,
        grid=(NT,),
        in_specs=[
            whole(S + 8, H),
            whole(3, H, D),
            whole(1, D),
            pl.BlockSpec((BT, D), lambda t: (t, 0)),
            pl.BlockSpec((BT, D), lambda t: (t, 0)),
            pl.BlockSpec((BT, D), lambda t: (t, 0)),
            whole(1, D),
            whole(1, D),
        ],
        out_specs=pl.BlockSpec((BT, D), lambda t: (t, 0)),
        out_shape=jax.ShapeDtypeStruct((S, D), jnp.float32),
    )(hpad, s2, sh2_b[None, :], xf, y0, y1, ln_g[None, :], ln_b[None, :])

    return out.reshape(1, S, D)
